# baseline (speedup 1.0000x reference)
import functools

import jax
import jax.numpy as jnp
from jax.experimental import pallas as pl
from jax.experimental.pallas import tpu as pltpu


def _row_gather_kernel(rows_ref, table_ref, out_ref, sem, *, rows_per_core):
    core = pl.program_id(0)
    base = core * rows_per_core
    for i in range(rows_per_core):
        row = rows_ref[base + i]
        pltpu.make_async_copy(
            table_ref.at[pl.ds(row, 1), :],
            out_ref.at[pl.ds(base + i, 1), :],
            sem,
        ).start()
    pltpu.make_async_copy(
        table_ref.at[pl.ds(0, rows_per_core), :],
        out_ref.at[pl.ds(0, rows_per_core), :],
        sem,
    ).wait()


def kernel(labels, table, force_drop_ids):
    B = labels.shape[0]
    V, H = table.shape
    num_classes = V - 1
    eff = jnp.where(force_drop_ids == 1, num_classes,
                    labels.astype(jnp.int32)).astype(jnp.int32)
    eff = jnp.clip(eff, 0, V - 1)

    n_cores = 2 if B % 2 == 0 else 1
    rows_per_core = B // n_cores

    return pl.pallas_call(
        functools.partial(_row_gather_kernel, rows_per_core=rows_per_core),
        grid=(n_cores,),
        in_specs=[
            pl.BlockSpec(memory_space=pltpu.SMEM),
            pl.BlockSpec(memory_space=pltpu.HBM),
        ],
        out_specs=pl.BlockSpec(memory_space=pltpu.HBM),
        out_shape=jax.ShapeDtypeStruct((B, H), table.dtype),
        scratch_shapes=[pltpu.SemaphoreType.DMA],
        compiler_params=pltpu.CompilerParams(
            dimension_semantics=("parallel",),
            disable_bounds_checks=True,
        ),
    )(eff, table)


# speedup vs baseline: 4.0644x; 4.0644x over previous
import functools

import jax
import jax.numpy as jnp
from jax.experimental import pallas as pl
from jax.experimental.pallas import tpu as pltpu


def _row_gather_kernel(rows_ref, table_ref, out_ref, sem, *, rows_per_core):
    core = pl.program_id(0)
    base = core * rows_per_core
    for i in range(rows_per_core):
        row = rows_ref[base + i]
        pltpu.make_async_copy(
            table_ref.at[pl.ds(row, 1), :],
            out_ref.at[i],
            sem,
        ).start()
    pltpu.make_async_copy(
        table_ref.at[pl.ds(0, rows_per_core), :],
        table_ref.at[pl.ds(0, rows_per_core), :],
        sem,
    ).wait()


def kernel(labels, table, force_drop_ids):
    B = labels.shape[0]
    V, H = table.shape
    num_classes = V - 1
    eff = jnp.where(force_drop_ids == 1, num_classes,
                    labels.astype(jnp.int32)).astype(jnp.int32)
    eff = jnp.clip(eff, 0, V - 1)

    n_cores = 2 if B % 2 == 0 else 1
    rows_per_core = B // n_cores

    out = pl.pallas_call(
        functools.partial(_row_gather_kernel, rows_per_core=rows_per_core),
        grid=(n_cores,),
        in_specs=[
            pl.BlockSpec(memory_space=pltpu.SMEM),
            pl.BlockSpec(memory_space=pltpu.HBM),
        ],
        out_specs=pl.BlockSpec((rows_per_core, 1, H), lambda i: (i, 0, 0)),
        out_shape=jax.ShapeDtypeStruct((B, 1, H), table.dtype),
        scratch_shapes=[pltpu.SemaphoreType.DMA],
        compiler_params=pltpu.CompilerParams(
            dimension_semantics=("parallel",),
            disable_bounds_checks=True,
        ),
    )(eff, table)
    return out.reshape(B, H)
